# baseline (speedup 1.0000x reference)
import functools

import jax
import jax.numpy as jnp
from jax import lax
from jax.experimental import pallas as pl
from jax.experimental.pallas import tpu as pltpu
from jax.experimental.pallas import tpu_sc as plsc

N = 10000
E = 320000
IN_DIM = 128
HID = 512
EPS = 1e-5

NCHUNK = 4
DC = HID // NCHUNK
NSC = 2
NTILE = 16
KB = 128

NBLK = 160
EPT = NBLK * KB
E_PAD = EPT * NTILE
NSTAGE = 4
NBLK_S = NBLK // NSTAGE
EPT_S = EPT // NSTAGE
NPAIR = NBLK_S // 2

NQ = E_PAD // 4
QPT = NQ // NTILE
MROWS = KB // 4
L_CAP = 49152
LPT = L_CAP // NTILE
LBLK = LPT // KB
LPAIR = LBLK // 2
JUNK = 112
ROWS_PT = 632
N_ACC = ROWS_PT * NTILE

EPT32 = E_PAD // (NSC * NTILE)
NC_PAD = 10240
CNT_COLS = NC_PAD // NTILE

_MESH = plsc.VectorSubcoreMesh(core_axis_name="c", subcore_axis_name="s")
_SC_PARAMS = pltpu.CompilerParams(needs_layout_passes=False)


def _segsum_body(htab, src4, mdst2d, lsrc4, ldst2d, zeros_hbm, out,
                 src_v, dstq_v, dstl_v, g_a, g_b, mbuf,
                 sem_a, sem_b, acc):
    cid = lax.axis_index("c")
    sid = lax.axis_index("s")

    def gather(idx_ref, j, buf, sem):
        return pltpu.make_async_copy(htab.at[idx_ref.at[pl.ds(j * KB, KB)]],
                                     buf, sem)

    def merge(gb):
        def mrow(m, carry):
            for g in range(DC // 16):
                sl = pl.ds(g * 16, 16)
                mbuf[m, sl] = (gb[4 * m, sl] + gb[4 * m + 1, sl]
                               + gb[4 * m + 2, sl] + gb[4 * m + 3, sl])
            return carry

        lax.fori_loop(0, MROWS, mrow, 0)

    for cc in range(2):
        cglob = cid * 2 + cc
        pltpu.sync_copy(zeros_hbm, acc.at[pl.ds(sid * ROWS_PT, ROWS_PT)])
        plsc.subcore_barrier()
        for q in range(NSTAGE):
            e0 = cglob * E_PAD + sid * EPT + q * EPT_S
            pltpu.sync_copy(src4.at[pl.ds(e0, EPT_S)], src_v)
            pltpu.sync_copy(
                mdst2d.at[pl.ds(sid * NBLK + q * NBLK_S, NBLK_S)], dstq_v)
            gather(src_v, 0, g_a, sem_a).start()
            gather(src_v, 1, g_b, sem_b).start()

            def pair(i, carry):
                j0 = 2 * i
                gather(src_v, j0, g_a, sem_a).wait()
                merge(g_a)

                @pl.when(i < NPAIR - 1)
                def _():
                    gather(src_v, j0 + 2, g_a, sem_a).start()

                pltpu.sync_copy(mbuf, acc.at[dstq_v.at[j0]], add=True)
                gather(src_v, j0 + 1, g_b, sem_b).wait()
                merge(g_b)

                @pl.when(i < NPAIR - 1)
                def _():
                    gather(src_v, j0 + 3, g_b, sem_b).start()

                pltpu.sync_copy(mbuf, acc.at[dstq_v.at[j0 + 1]], add=True)
                return carry

            lax.fori_loop(0, NPAIR, pair, 0)

        l0 = cglob * L_CAP + sid * LPT
        pltpu.sync_copy(lsrc4.at[pl.ds(l0, LPT)], src_v.at[pl.ds(0, LPT)])
        pltpu.sync_copy(ldst2d.at[pl.ds(sid * LBLK, LBLK)], dstl_v)
        gather(src_v, 0, g_a, sem_a).start()
        gather(src_v, 1, g_b, sem_b).start()

        def lpair(i, carry):
            j0 = 2 * i
            gather(src_v, j0, g_a, sem_a).wait()
            pltpu.sync_copy(g_a, acc.at[dstl_v.at[j0]], add=True)

            @pl.when(i < LPAIR - 1)
            def _():
                gather(src_v, j0 + 2, g_a, sem_a).start()

            gather(src_v, j0 + 1, g_b, sem_b).wait()
            pltpu.sync_copy(g_b, acc.at[dstl_v.at[j0 + 1]], add=True)

            @pl.when(i < LPAIR - 1)
            def _():
                gather(src_v, j0 + 3, g_b, sem_b).start()

            return carry

        lax.fori_loop(0, LPAIR, lpair, 0)
        plsc.subcore_barrier()
        r0 = sid * ROWS_PT
        pltpu.sync_copy(acc.at[pl.ds(r0, ROWS_PT)],
                        out.at[cglob, pl.ds(r0, ROWS_PT)])
        plsc.subcore_barrier()


_segsum_kernel = pl.kernel(
    _segsum_body,
    out_type=jax.ShapeDtypeStruct((NCHUNK, N_ACC, DC), jnp.float32),
    mesh=_MESH,
    scratch_types=[
        pltpu.VMEM((EPT_S,), jnp.int32),
        pltpu.VMEM((NBLK_S, MROWS), jnp.int32),
        pltpu.VMEM((LBLK, KB), jnp.int32),
        pltpu.VMEM((KB, DC), jnp.float32),
        pltpu.VMEM((KB, DC), jnp.float32),
        pltpu.VMEM((MROWS, DC), jnp.float32),
        pltpu.SemaphoreType.DMA,
        pltpu.SemaphoreType.DMA,
        pltpu.VMEM_SHARED((N_ACC, DC), jnp.float32),
    ],
    compiler_params=_SC_PARAMS,
)


def _cnt_body(dst_hbm, out, local, dstbuf, outbuf, shr):
    cid = lax.axis_index("c")
    sid = lax.axis_index("s")
    zero16 = jnp.zeros((16,), jnp.float32)
    ones16 = jnp.full((16,), 1.0, jnp.float32)

    def zrow(i, carry):
        local[pl.ds(i * 16, 16)] = zero16
        return carry

    lax.fori_loop(0, NC_PAD // 16, zrow, 0)

    g = cid * NTILE + sid
    pltpu.sync_copy(dst_hbm.at[pl.ds(g * EPT32, EPT32)], dstbuf)

    def eb(i, carry):
        dv = dstbuf[pl.ds(i * 16, 16)]
        plsc.addupdate_scatter(local, [dv], ones16)
        return carry

    lax.fori_loop(0, EPT32 // 16, eb, 0)

    pltpu.sync_copy(local, shr.at[pl.ds(sid * NC_PAD, NC_PAD)])
    plsc.subcore_barrier()

    for r in range(NTILE):
        pltpu.sync_copy(shr.at[pl.ds(r * NC_PAD + sid * CNT_COLS, CNT_COLS)],
                        local.at[pl.ds(r * CNT_COLS, CNT_COLS)])

    def red(i, carry):
        s = local[pl.ds(i * 16, 16)]
        for r in range(1, NTILE):
            s = s + local[pl.ds(r * CNT_COLS + i * 16, 16)]
        outbuf[pl.ds(i * 16, 16)] = s
        return carry

    lax.fori_loop(0, CNT_COLS // 16, red, 0)
    pltpu.sync_copy(outbuf, out.at[pl.ds(cid * NC_PAD + sid * CNT_COLS, CNT_COLS)])


_cnt_kernel = pl.kernel(
    _cnt_body,
    out_type=jax.ShapeDtypeStruct((NSC * NC_PAD,), jnp.float32),
    mesh=_MESH,
    scratch_types=[
        pltpu.VMEM((NC_PAD,), jnp.float32),
        pltpu.VMEM((EPT32,), jnp.int32),
        pltpu.VMEM((CNT_COLS,), jnp.float32),
        pltpu.VMEM_SHARED((NTILE * NC_PAD,), jnp.float32),
    ],
    compiler_params=_SC_PARAMS,
)


BN = 2000


def _chunk_major(h):
    return jnp.stack([h[:, c * DC:(c + 1) * DC] for c in range(NCHUNK)], axis=0)


def _in_proj_body(x_ref, w_ref, b_ref, o1_ref, o2_ref):
    h = jnp.dot(x_ref[...], w_ref[...], preferred_element_type=jnp.float32)
    h = jnp.maximum(h + b_ref[...], 0.0)
    o1_ref[...] = h
    o2_ref[...] = _chunk_major(h)


_in_proj = pl.pallas_call(
    _in_proj_body,
    grid=(N // BN,),
    in_specs=[
        pl.BlockSpec((BN, IN_DIM), lambda i: (i, 0)),
        pl.BlockSpec((IN_DIM, HID), lambda i: (0, 0)),
        pl.BlockSpec((1, HID), lambda i: (0, 0)),
    ],
    out_specs=[
        pl.BlockSpec((BN, HID), lambda i: (i, 0)),
        pl.BlockSpec((NCHUNK, BN, DC), lambda i: (0, i, 0)),
    ],
    out_shape=[
        jax.ShapeDtypeStruct((N, HID), jnp.float32),
        jax.ShapeDtypeStruct((NCHUNK, N, DC), jnp.float32),
    ],
)


def _layer_body(h_ref, ng_ref, ca_ref, cb_ref, ws_ref, wn_ref, b_ref, g_ref,
                be_ref, o1_ref, o2_ref):
    h = h_ref[...]
    ng = jnp.concatenate([ng_ref[c] for c in range(NCHUNK)], axis=1)
    denom = jnp.maximum(ca_ref[...] + cb_ref[...], 1.0)
    nm = ng / denom
    hh = (jnp.dot(h, ws_ref[...], preferred_element_type=jnp.float32)
          + jnp.dot(nm, wn_ref[...], preferred_element_type=jnp.float32)
          + b_ref[...])
    m = jnp.mean(hh, axis=-1, keepdims=True)
    v = jnp.mean((hh - m) ** 2, axis=-1, keepdims=True)
    hh = (hh - m) / jnp.sqrt(v + EPS) * g_ref[...] + be_ref[...]
    hn = h + jnp.maximum(hh, 0.0)
    o1_ref[...] = hn
    o2_ref[...] = _chunk_major(hn)


_layer = pl.pallas_call(
    _layer_body,
    grid=(N // BN,),
    in_specs=[
        pl.BlockSpec((BN, HID), lambda i: (i, 0)),
        pl.BlockSpec((NCHUNK, BN, DC), lambda i: (0, i, 0)),
        pl.BlockSpec((BN, 1), lambda i: (i, 0)),
        pl.BlockSpec((BN, 1), lambda i: (i, 0)),
        pl.BlockSpec((HID, HID), lambda i: (0, 0)),
        pl.BlockSpec((HID, HID), lambda i: (0, 0)),
        pl.BlockSpec((1, HID), lambda i: (0, 0)),
        pl.BlockSpec((1, HID), lambda i: (0, 0)),
        pl.BlockSpec((1, HID), lambda i: (0, 0)),
    ],
    out_specs=[
        pl.BlockSpec((BN, HID), lambda i: (i, 0)),
        pl.BlockSpec((NCHUNK, BN, DC), lambda i: (0, i, 0)),
    ],
    out_shape=[
        jax.ShapeDtypeStruct((N, HID), jnp.float32),
        jax.ShapeDtypeStruct((NCHUNK, N, DC), jnp.float32),
    ],
)


def kernel(x, edge_index, deg, W_in, b_in,
           W_self_0, W_neigh_0, bias_0, gamma_0, beta_0,
           W_self_1, W_neigh_1, bias_1, gamma_1, beta_1,
           W_self_2, W_neigh_2, bias_2, gamma_2, beta_2):
    del deg
    src = edge_index[0].astype(jnp.int32)
    dst = edge_index[1].astype(jnp.int32)
    pad = E_PAD - E
    src_p = jnp.concatenate([src, jnp.zeros((pad,), jnp.int32)])
    dst_p = jnp.concatenate([dst, jnp.full((pad,), N, jnp.int32)])
    dst_p, src_p = jax.lax.sort((dst_p, src_p), num_keys=1)
    offs = (jnp.arange(NCHUNK, dtype=jnp.int32) * N)[:, None]
    src4 = (src_p[None, :] + offs).reshape(-1)
    qd = dst_p.reshape(NQ, 4)
    ok = qd[:, 0] == qd[:, 3]
    qidx = jnp.arange(NQ, dtype=jnp.int32)
    mdst2d = jnp.where(ok, qd[:, 0],
                       N + (qidx % JUNK)).reshape(NQ // MROWS, MROWS)
    mask = jnp.repeat(~ok, 4)
    tgt = jnp.where(mask, jnp.cumsum(mask) - 1, L_CAP)
    lidx = jnp.arange(L_CAP, dtype=jnp.int32)
    lsrc = jnp.zeros((L_CAP,), jnp.int32).at[tgt].set(src_p, mode="drop")
    ldst = (N + (lidx % JUNK)).at[tgt].set(dst_p, mode="drop")
    lsrc4 = (lsrc[None, :] + offs).reshape(-1)
    ldst2d = ldst.reshape(L_CAP // KB, KB)
    zeros_rows = jnp.zeros((ROWS_PT, DC), jnp.float32)

    cnt2 = _cnt_kernel(dst_p).reshape(NSC, NC_PAD)
    ca = cnt2[0, :N, None]
    cb = cnt2[1, :N, None]

    h, htab = _in_proj(x, W_in, b_in.reshape(1, HID))
    for (Ws, Wn, bb, gg, be) in (
            (W_self_0, W_neigh_0, bias_0, gamma_0, beta_0),
            (W_self_1, W_neigh_1, bias_1, gamma_1, beta_1),
            (W_self_2, W_neigh_2, bias_2, gamma_2, beta_2)):
        ng = _segsum_kernel(htab.reshape(NCHUNK * N, DC), src4, mdst2d,
                            lsrc4, ldst2d, zeros_rows)[:, :N]
        h, htab = _layer(h, ng, ca, cb, Ws, Wn, bb.reshape(1, HID),
                         gg.reshape(1, HID), be.reshape(1, HID))
    return h


# speedup vs baseline: 3.0745x; 3.0745x over previous
import functools

import jax
import jax.numpy as jnp
from jax import lax
from jax.experimental import pallas as pl
from jax.experimental.pallas import tpu as pltpu
from jax.experimental.pallas import tpu_sc as plsc

N = 10000
E = 320000
IN_DIM = 128
HID = 512
EPS = 1e-5

NCHUNK = 4
DC = HID // NCHUNK
NSC = 2
NTILE = 16
KB = 128

NBLK = 160
EPT = NBLK * KB
E_PAD = EPT * NTILE
NSTAGE = 4
NBLK_S = NBLK // NSTAGE
EPT_S = EPT // NSTAGE
NPAIR = NBLK_S // 2
ROWS_PT = 632
N_ACC = ROWS_PT * NTILE

EPT32 = E_PAD // (NSC * NTILE)
NC_PAD = 10240
CNT_COLS = NC_PAD // NTILE

_MESH = plsc.VectorSubcoreMesh(core_axis_name="c", subcore_axis_name="s")
_SC_PARAMS = pltpu.CompilerParams(needs_layout_passes=False)


def _segsum_body(htab, src4, dst2d, zeros_hbm, out, src_v, dst_v, g_a, g_b,
                 sem_a, sem_b, acc):
    cid = lax.axis_index("c")
    sid = lax.axis_index("s")

    def gather(j, buf, sem):
        return pltpu.make_async_copy(htab.at[src_v.at[pl.ds(j * KB, KB)]],
                                     buf, sem)

    for cc in range(2):
        cglob = cid * 2 + cc
        pltpu.sync_copy(zeros_hbm, acc.at[pl.ds(sid * ROWS_PT, ROWS_PT)])
        plsc.subcore_barrier()
        for q in range(NSTAGE):
            e0 = cglob * E_PAD + sid * EPT + q * EPT_S
            pltpu.sync_copy(src4.at[pl.ds(e0, EPT_S)], src_v)
            pltpu.sync_copy(dst2d.at[pl.ds(sid * NBLK + q * NBLK_S, NBLK_S)],
                            dst_v)
            gather(0, g_a, sem_a).start()
            gather(1, g_b, sem_b).start()

            def pair(i, carry):
                j0 = 2 * i
                gather(j0, g_a, sem_a).wait()
                pltpu.sync_copy(g_a, acc.at[dst_v.at[j0]], add=True)

                @pl.when(i < NPAIR - 1)
                def _():
                    gather(j0 + 2, g_a, sem_a).start()

                gather(j0 + 1, g_b, sem_b).wait()
                pltpu.sync_copy(g_b, acc.at[dst_v.at[j0 + 1]], add=True)

                @pl.when(i < NPAIR - 1)
                def _():
                    gather(j0 + 3, g_b, sem_b).start()

                return carry

            lax.fori_loop(0, NPAIR, pair, 0)
        plsc.subcore_barrier()
        r0 = sid * ROWS_PT
        pltpu.sync_copy(acc.at[pl.ds(r0, ROWS_PT)],
                        out.at[cglob, pl.ds(r0, ROWS_PT)])
        plsc.subcore_barrier()


_segsum_kernel = pl.kernel(
    _segsum_body,
    out_type=jax.ShapeDtypeStruct((NCHUNK, N_ACC, DC), jnp.float32),
    mesh=_MESH,
    scratch_types=[
        pltpu.VMEM((EPT_S,), jnp.int32),
        pltpu.VMEM((NBLK_S, KB), jnp.int32),
        pltpu.VMEM((KB, DC), jnp.float32),
        pltpu.VMEM((KB, DC), jnp.float32),
        pltpu.SemaphoreType.DMA,
        pltpu.SemaphoreType.DMA,
        pltpu.VMEM_SHARED((N_ACC, DC), jnp.float32),
    ],
    compiler_params=_SC_PARAMS,
)


def _cnt_body(dst_hbm, out, local, dstbuf, outbuf, shr):
    cid = lax.axis_index("c")
    sid = lax.axis_index("s")
    zero16 = jnp.zeros((16,), jnp.float32)
    ones16 = jnp.full((16,), 1.0, jnp.float32)

    def zrow(i, carry):
        local[pl.ds(i * 16, 16)] = zero16
        return carry

    lax.fori_loop(0, NC_PAD // 16, zrow, 0)

    g = cid * NTILE + sid
    pltpu.sync_copy(dst_hbm.at[pl.ds(g * EPT32, EPT32)], dstbuf)

    def eb(i, carry):
        dv = dstbuf[pl.ds(i * 16, 16)]
        plsc.addupdate_scatter(local, [dv], ones16)
        return carry

    lax.fori_loop(0, EPT32 // 16, eb, 0)

    pltpu.sync_copy(local, shr.at[pl.ds(sid * NC_PAD, NC_PAD)])
    plsc.subcore_barrier()

    for r in range(NTILE):
        pltpu.sync_copy(shr.at[pl.ds(r * NC_PAD + sid * CNT_COLS, CNT_COLS)],
                        local.at[pl.ds(r * CNT_COLS, CNT_COLS)])

    def red(i, carry):
        s = local[pl.ds(i * 16, 16)]
        for r in range(1, NTILE):
            s = s + local[pl.ds(r * CNT_COLS + i * 16, 16)]
        outbuf[pl.ds(i * 16, 16)] = s
        return carry

    lax.fori_loop(0, CNT_COLS // 16, red, 0)
    pltpu.sync_copy(outbuf, out.at[pl.ds(cid * NC_PAD + sid * CNT_COLS, CNT_COLS)])


_cnt_kernel = pl.kernel(
    _cnt_body,
    out_type=jax.ShapeDtypeStruct((NSC * NC_PAD,), jnp.float32),
    mesh=_MESH,
    scratch_types=[
        pltpu.VMEM((NC_PAD,), jnp.float32),
        pltpu.VMEM((EPT32,), jnp.int32),
        pltpu.VMEM((CNT_COLS,), jnp.float32),
        pltpu.VMEM_SHARED((NTILE * NC_PAD,), jnp.float32),
    ],
    compiler_params=_SC_PARAMS,
)


BN = 2000


def _chunk_major(h):
    return jnp.stack([h[:, c * DC:(c + 1) * DC] for c in range(NCHUNK)], axis=0)


def _in_proj_body(x_ref, w_ref, b_ref, o1_ref, o2_ref):
    h = jnp.dot(x_ref[...], w_ref[...], preferred_element_type=jnp.float32)
    h = jnp.maximum(h + b_ref[...], 0.0)
    o1_ref[...] = h
    o2_ref[...] = _chunk_major(h)


_in_proj = pl.pallas_call(
    _in_proj_body,
    grid=(N // BN,),
    in_specs=[
        pl.BlockSpec((BN, IN_DIM), lambda i: (i, 0)),
        pl.BlockSpec((IN_DIM, HID), lambda i: (0, 0)),
        pl.BlockSpec((1, HID), lambda i: (0, 0)),
    ],
    out_specs=[
        pl.BlockSpec((BN, HID), lambda i: (i, 0)),
        pl.BlockSpec((NCHUNK, BN, DC), lambda i: (0, i, 0)),
    ],
    out_shape=[
        jax.ShapeDtypeStruct((N, HID), jnp.float32),
        jax.ShapeDtypeStruct((NCHUNK, N, DC), jnp.float32),
    ],
)


def _layer_body(h_ref, ng_ref, ca_ref, cb_ref, ws_ref, wn_ref, b_ref, g_ref,
                be_ref, o1_ref, o2_ref):
    h = h_ref[...]
    ng = jnp.concatenate([ng_ref[c] for c in range(NCHUNK)], axis=1)
    denom = jnp.maximum(ca_ref[...] + cb_ref[...], 1.0)
    nm = ng / denom
    hh = (jnp.dot(h, ws_ref[...], preferred_element_type=jnp.float32)
          + jnp.dot(nm, wn_ref[...], preferred_element_type=jnp.float32)
          + b_ref[...])
    m = jnp.mean(hh, axis=-1, keepdims=True)
    v = jnp.mean((hh - m) ** 2, axis=-1, keepdims=True)
    hh = (hh - m) / jnp.sqrt(v + EPS) * g_ref[...] + be_ref[...]
    hn = h + jnp.maximum(hh, 0.0)
    o1_ref[...] = hn
    o2_ref[...] = _chunk_major(hn)


_layer = pl.pallas_call(
    _layer_body,
    grid=(N // BN,),
    in_specs=[
        pl.BlockSpec((BN, HID), lambda i: (i, 0)),
        pl.BlockSpec((NCHUNK, BN, DC), lambda i: (0, i, 0)),
        pl.BlockSpec((BN, 1), lambda i: (i, 0)),
        pl.BlockSpec((BN, 1), lambda i: (i, 0)),
        pl.BlockSpec((HID, HID), lambda i: (0, 0)),
        pl.BlockSpec((HID, HID), lambda i: (0, 0)),
        pl.BlockSpec((1, HID), lambda i: (0, 0)),
        pl.BlockSpec((1, HID), lambda i: (0, 0)),
        pl.BlockSpec((1, HID), lambda i: (0, 0)),
    ],
    out_specs=[
        pl.BlockSpec((BN, HID), lambda i: (i, 0)),
        pl.BlockSpec((NCHUNK, BN, DC), lambda i: (0, i, 0)),
    ],
    out_shape=[
        jax.ShapeDtypeStruct((N, HID), jnp.float32),
        jax.ShapeDtypeStruct((NCHUNK, N, DC), jnp.float32),
    ],
)


def kernel(x, edge_index, deg, W_in, b_in,
           W_self_0, W_neigh_0, bias_0, gamma_0, beta_0,
           W_self_1, W_neigh_1, bias_1, gamma_1, beta_1,
           W_self_2, W_neigh_2, bias_2, gamma_2, beta_2):
    del deg
    src = edge_index[0].astype(jnp.int32)
    dst = edge_index[1].astype(jnp.int32)
    pad = E_PAD - E
    src_p = jnp.concatenate([src, jnp.zeros((pad,), jnp.int32)])
    dst_p = jnp.concatenate([dst, jnp.full((pad,), N, jnp.int32)])
    src4 = (src_p[None, :]
            + (jnp.arange(NCHUNK, dtype=jnp.int32) * N)[:, None]).reshape(-1)
    dst2d = dst_p.reshape(E_PAD // KB, KB)
    zeros_rows = jnp.zeros((ROWS_PT, DC), jnp.float32)

    cnt2 = _cnt_kernel(dst_p).reshape(NSC, NC_PAD)
    ca = cnt2[0, :N, None]
    cb = cnt2[1, :N, None]

    h, htab = _in_proj(x, W_in, b_in.reshape(1, HID))
    for (Ws, Wn, bb, gg, be) in (
            (W_self_0, W_neigh_0, bias_0, gamma_0, beta_0),
            (W_self_1, W_neigh_1, bias_1, gamma_1, beta_1),
            (W_self_2, W_neigh_2, bias_2, gamma_2, beta_2)):
        ng = _segsum_kernel(htab.reshape(NCHUNK * N, DC), src4, dst2d,
                            zeros_rows)[:, :N]
        h, htab = _layer(h, ng, ca, cb, Ws, Wn, bb.reshape(1, HID),
                         gg.reshape(1, HID), be.reshape(1, HID))
    return h
